# baseline (speedup 1.0000x reference)
import functools

import jax
import jax.numpy as jnp
from jax import lax
from jax.experimental import pallas as pl
from jax.experimental.pallas import tpu as pltpu
from jax.experimental.pallas import tpu_sc as plsc

N = 10000
D = 128
E = 320000
NC = 2
NS = 16
NW = NC * NS
CHUNK = 96
NCHUNK0 = 140
NCHUNK1 = 72
EPT0 = CHUNK * NCHUNK0
EPT1 = CHUNK * NCHUNK1
E_PAD = NS * (EPT0 + EPT1)
N_ACC = N + 8
ZT = 10
ZR = N // ZT
ND_PAD = 10240
DPT = ND_PAD // NS


def _sc_mesh():
    return plsc.VectorSubcoreMesh(core_axis_name="c", subcore_axis_name="s")


def _sc_degree(dst1d, zdeg):
    @functools.partial(
        pl.kernel,
        out_type=jax.ShapeDtypeStruct((NC * ND_PAD,), jnp.float32),
        mesh=_sc_mesh(),
        scratch_types=[
            [pltpu.VMEM((CHUNK,), jnp.int32) for _ in range(4)],
            pltpu.VMEM((CHUNK,), jnp.float32),
            pltpu.VMEM_SHARED((ND_PAD,), jnp.float32),
            pltpu.SemaphoreType.DMA,
            pltpu.SemaphoreType.DMA,
        ],
    )
    def run(dst_hbm, z_hbm, out_hbm, dbufs, ones, acc, isem, ssem):
        c = lax.axis_index("c")
        s = lax.axis_index("s")
        base = jnp.where(c == 0, s * EPT0, NS * EPT0 + s * EPT1)
        ngrp = jnp.where(c == 0, NCHUNK0 // 4, NCHUNK1 // 4)
        for k in range(4):
            pltpu.async_copy(dst_hbm.at[pl.ds(base + k * CHUNK, CHUNK)],
                             dbufs[k], isem)
        pltpu.sync_copy(z_hbm.at[pl.ds(c * ND_PAD + s * DPT, DPT)],
                        acc.at[pl.ds(s * DPT, DPT)])
        for j in range(CHUNK // 16):
            ones[pl.ds(j * 16, 16)] = jnp.full((16,), 1.0, jnp.float32)
        plsc.subcore_barrier()

        @pl.loop(0, ngrp)
        def _(j):
            b = base + (4 * j) * CHUNK
            for k in range(4):
                pltpu.make_async_copy(dst_hbm.at[pl.ds(b + k * CHUNK, CHUNK)],
                                      dbufs[k], isem).wait()
                pltpu.async_copy(ones, acc.at[dbufs[k]], ssem, add=True)
            for k in range(4):
                pltpu.make_async_copy(ones, acc.at[dbufs[k]], ssem).wait()

            @pl.when(j + 1 < ngrp)
            def _():
                for k in range(4):
                    pltpu.async_copy(
                        dst_hbm.at[pl.ds(b + (4 + k) * CHUNK, CHUNK)],
                        dbufs[k], isem)

        plsc.subcore_barrier()
        pltpu.sync_copy(acc.at[pl.ds(s * DPT, DPT)],
                        out_hbm.at[pl.ds(c * ND_PAD + s * DPT, DPT)])

    return run(dst1d, zdeg)


def _sc_aggregate(g, src1d, dst1d, zrows):
    @functools.partial(
        pl.kernel,
        out_type=jax.ShapeDtypeStruct((NC * N, D), jnp.float32),
        mesh=_sc_mesh(),
        scratch_types=[
            pltpu.VMEM((CHUNK,), jnp.int32),
            pltpu.VMEM((CHUNK,), jnp.int32),
            pltpu.VMEM((CHUNK,), jnp.int32),
            pltpu.VMEM((CHUNK,), jnp.int32),
            pltpu.VMEM((CHUNK, D), jnp.float32),
            pltpu.VMEM((CHUNK, D), jnp.float32),
            pltpu.VMEM_SHARED((N_ACC, D), jnp.float32),
            pltpu.SemaphoreType.DMA,
            pltpu.SemaphoreType.DMA,
            pltpu.SemaphoreType.DMA,
            pltpu.SemaphoreType.DMA,
        ],
    )
    def run(g_hbm, src_hbm, dst_hbm, z_hbm, out_hbm,
            sidx0, didx0, sidx1, didx1, rows0, rows1, acc, is0, is1, gs0, gs1):
        c = lax.axis_index("c")
        s = lax.axis_index("s")
        base = jnp.where(c == 0, s * EPT0, NS * EPT0 + s * EPT1)
        nchunk = jnp.where(c == 0, NCHUNK0, NCHUNK1)

        def load_idx(b, sbuf, dbuf, sem):
            pltpu.async_copy(src_hbm.at[pl.ds(b, CHUNK)], sbuf, sem)
            pltpu.async_copy(dst_hbm.at[pl.ds(b, CHUNK)], dbuf, sem)

        def wait_idx(b, sbuf, dbuf, sem):
            pltpu.make_async_copy(src_hbm.at[pl.ds(b, CHUNK)], sbuf, sem).wait()
            pltpu.make_async_copy(dst_hbm.at[pl.ds(b, CHUNK)], dbuf, sem).wait()

        load_idx(base, sidx0, didx0, is0)
        wait_idx(base, sidx0, didx0, is0)
        pltpu.async_copy(g_hbm.at[sidx0], rows0, gs0)
        load_idx(base + CHUNK, sidx1, didx1, is1)

        @pl.when(s < ZT)
        def _():
            pltpu.sync_copy(z_hbm.at[pl.ds(c * N + s * ZR, ZR)],
                            acc.at[pl.ds(s * ZR, ZR)])

        plsc.subcore_barrier()

        @pl.loop(0, nchunk // 2)
        def _(j):
            b0 = base + (2 * j) * CHUNK
            b1 = b0 + CHUNK
            wait_idx(b1, sidx1, didx1, is1)
            pltpu.async_copy(g_hbm.at[sidx1], rows1, gs1)
            pltpu.make_async_copy(g_hbm.at[sidx0], rows0, gs0).wait()
            pltpu.sync_copy(rows0, acc.at[didx0], add=True)

            @pl.when(2 * j + 2 < nchunk)
            def _():
                load_idx(b0 + 2 * CHUNK, sidx0, didx0, is0)

            pltpu.make_async_copy(g_hbm.at[sidx1], rows1, gs1).wait()
            pltpu.sync_copy(rows1, acc.at[didx1], add=True)

            @pl.when(2 * j + 3 < nchunk)
            def _():
                load_idx(b1 + 2 * CHUNK, sidx1, didx1, is1)

            @pl.when(2 * j + 2 < nchunk)
            def _():
                wait_idx(b0 + 2 * CHUNK, sidx0, didx0, is0)
                pltpu.async_copy(g_hbm.at[sidx0], rows0, gs0)

        plsc.subcore_barrier()

        @pl.when(s < ZT)
        def _():
            pltpu.sync_copy(acc.at[pl.ds(s * ZR, ZR)],
                            out_hbm.at[pl.ds(c * N + s * ZR, ZR)])

    return run(g, src1d, dst1d, zrows)


R = 1000
G = N // R


def _row_spec():
    return pl.BlockSpec((R, D), lambda i: (i, 0))


def _deg_spec():
    return pl.BlockSpec((R, 2), lambda i: (i, 0))


def _w_spec():
    return pl.BlockSpec((D, D), lambda i: (0, 0))


def _b_spec():
    return pl.BlockSpec((1, D), lambda i: (0, 0))


def _dinv_of(deg_blk):
    return lax.rsqrt(deg_blk[:, 0] + deg_blk[:, 1] + 1.0)


def _tc_pre(deg2, x, w):
    def body(deg_ref, x_ref, w_ref, g_ref):
        dinv = _dinv_of(deg_ref[...])
        h = jnp.dot(x_ref[...], w_ref[...], preferred_element_type=jnp.float32)
        g_ref[...] = h * dinv[:, None]

    return pl.pallas_call(
        body,
        grid=(G,),
        in_specs=[_deg_spec(), _row_spec(), _w_spec()],
        out_specs=_row_spec(),
        out_shape=jax.ShapeDtypeStruct((N, D), jnp.float32),
    )(deg2, x, w)


def _tc_mid(deg2, parts, gprev, b, w):
    def body(deg_ref, p0_ref, p1_ref, gp_ref, b_ref, w_ref, g_ref):
        dinv = _dinv_of(deg_ref[...])
        t = (p0_ref[...] + p1_ref[...] + gp_ref[...]) * dinv[:, None] + b_ref[...]
        z = jnp.maximum(t, 0.0)
        h = jnp.dot(z, w_ref[...], preferred_element_type=jnp.float32)
        g_ref[...] = h * dinv[:, None]

    p1_spec = pl.BlockSpec((R, D), lambda i: (G + i, 0))
    return pl.pallas_call(
        body,
        grid=(G,),
        in_specs=[_deg_spec(), _row_spec(), p1_spec, _row_spec(), _b_spec(), _w_spec()],
        out_specs=_row_spec(),
        out_shape=jax.ShapeDtypeStruct((N, D), jnp.float32),
    )(deg2, parts, parts, gprev, b, w)


def _tc_post(deg2, parts, g3, b):
    def body(deg_ref, p0_ref, p1_ref, g_ref, b_ref, o_ref):
        dinv = _dinv_of(deg_ref[...])
        o_ref[...] = (p0_ref[...] + p1_ref[...] + g_ref[...]) * dinv[:, None] + b_ref[...]

    p1_spec = pl.BlockSpec((R, D), lambda i: (G + i, 0))
    return pl.pallas_call(
        body,
        grid=(G,),
        in_specs=[_deg_spec(), _row_spec(), p1_spec, _row_spec(), _b_spec()],
        out_specs=_row_spec(),
        out_shape=jax.ShapeDtypeStruct((N, D), jnp.float32),
    )(deg2, parts, parts, g3, b)


def kernel(x, edge_index, W1, b1, W2, b2, W3, b3):
    ei = edge_index.astype(jnp.int32)
    npad = E_PAD - E
    src1d = jnp.concatenate([ei[0], jnp.zeros((npad,), jnp.int32)])
    dst1d = jnp.concatenate([ei[1], jnp.full((npad,), N, jnp.int32)])
    zdeg = jnp.zeros((NC * ND_PAD,), jnp.float32)
    zrows = jnp.zeros((NC * N, D), jnp.float32)

    degp = _sc_degree(dst1d, zdeg)
    deg2 = degp.reshape(NC, ND_PAD)[:, :N].T

    b1r = b1.reshape(1, D)
    b2r = b2.reshape(1, D)
    b3r = b3.reshape(1, D)

    g1 = _tc_pre(deg2, x, W1)
    p = _sc_aggregate(g1, src1d, dst1d, zrows)
    g2 = _tc_mid(deg2, p, g1, b1r, W2)
    q = _sc_aggregate(g2, src1d, dst1d, zrows)
    g3 = _tc_mid(deg2, q, g2, b2r, W3)
    r = _sc_aggregate(g3, src1d, dst1d, zrows)
    return _tc_post(deg2, r, g3, b3r)


# speedup vs baseline: 1.2500x; 1.2500x over previous
import functools

import jax
import jax.numpy as jnp
from jax import lax
from jax.experimental import pallas as pl
from jax.experimental.pallas import tpu as pltpu
from jax.experimental.pallas import tpu_sc as plsc

N = 10000
D = 128
E = 320000
NC = 2
NS = 16
NW = NC * NS
CHUNK = 64
NCHUNK0 = 204
NCHUNK1 = 112
EPT0 = CHUNK * NCHUNK0
EPT1 = CHUNK * NCHUNK1
E_PAD = NS * (EPT0 + EPT1)
N_ACC = N + 8
ZT = 10
ZR = N // ZT
ND_PAD = 10240
DPT = ND_PAD // NS


def _sc_mesh():
    return plsc.VectorSubcoreMesh(core_axis_name="c", subcore_axis_name="s")


def _sc_degree(dst1d, zdeg):
    @functools.partial(
        pl.kernel,
        out_type=jax.ShapeDtypeStruct((NC * ND_PAD,), jnp.float32),
        mesh=_sc_mesh(),
        scratch_types=[
            [pltpu.VMEM((CHUNK,), jnp.int32) for _ in range(4)],
            pltpu.VMEM((CHUNK,), jnp.float32),
            pltpu.VMEM_SHARED((ND_PAD,), jnp.float32),
            pltpu.SemaphoreType.DMA,
            pltpu.SemaphoreType.DMA,
        ],
    )
    def run(dst_hbm, z_hbm, out_hbm, dbufs, ones, acc, isem, ssem):
        c = lax.axis_index("c")
        s = lax.axis_index("s")
        base = jnp.where(c == 0, s * EPT0, NS * EPT0 + s * EPT1)
        ngrp = jnp.where(c == 0, NCHUNK0 // 4, NCHUNK1 // 4)
        for k in range(4):
            pltpu.async_copy(dst_hbm.at[pl.ds(base + k * CHUNK, CHUNK)],
                             dbufs[k], isem)
        pltpu.sync_copy(z_hbm.at[pl.ds(c * ND_PAD + s * DPT, DPT)],
                        acc.at[pl.ds(s * DPT, DPT)])
        for j in range(CHUNK // 16):
            ones[pl.ds(j * 16, 16)] = jnp.full((16,), 1.0, jnp.float32)
        plsc.subcore_barrier()

        @pl.loop(0, ngrp)
        def _(j):
            b = base + (4 * j) * CHUNK
            for k in range(4):
                pltpu.make_async_copy(dst_hbm.at[pl.ds(b + k * CHUNK, CHUNK)],
                                      dbufs[k], isem).wait()
                pltpu.async_copy(ones, acc.at[dbufs[k]], ssem, add=True)
            for k in range(4):
                pltpu.make_async_copy(ones, acc.at[dbufs[k]], ssem).wait()

            @pl.when(j + 1 < ngrp)
            def _():
                for k in range(4):
                    pltpu.async_copy(
                        dst_hbm.at[pl.ds(b + (4 + k) * CHUNK, CHUNK)],
                        dbufs[k], isem)

        plsc.subcore_barrier()
        pltpu.sync_copy(acc.at[pl.ds(s * DPT, DPT)],
                        out_hbm.at[pl.ds(c * ND_PAD + s * DPT, DPT)])

    return run(dst1d, zdeg)


def _sc_aggregate(g, src1d, dst1d, zrows):
    @functools.partial(
        pl.kernel,
        out_type=jax.ShapeDtypeStruct((NC * N, D), jnp.float32),
        mesh=_sc_mesh(),
        scratch_types=[
            pltpu.VMEM((CHUNK,), jnp.int32),
            pltpu.VMEM((CHUNK,), jnp.int32),
            pltpu.VMEM((CHUNK,), jnp.int32),
            pltpu.VMEM((CHUNK,), jnp.int32),
            pltpu.VMEM((CHUNK, D), jnp.float32),
            pltpu.VMEM((CHUNK, D), jnp.float32),
            pltpu.VMEM_SHARED((N_ACC, D), jnp.float32),
            pltpu.SemaphoreType.DMA,
            pltpu.SemaphoreType.DMA,
            pltpu.SemaphoreType.DMA,
            pltpu.SemaphoreType.DMA,
        ],
    )
    def run(g_hbm, src_hbm, dst_hbm, z_hbm, out_hbm,
            sidx0, didx0, sidx1, didx1, rows0, rows1, acc, is0, is1, gs0, gs1):
        c = lax.axis_index("c")
        s = lax.axis_index("s")
        base = jnp.where(c == 0, s * EPT0, NS * EPT0 + s * EPT1)
        nchunk = jnp.where(c == 0, NCHUNK0, NCHUNK1)

        def load_idx(b, sbuf, dbuf, sem):
            pltpu.async_copy(src_hbm.at[pl.ds(b, CHUNK)], sbuf, sem)
            pltpu.async_copy(dst_hbm.at[pl.ds(b, CHUNK)], dbuf, sem)

        def wait_idx(b, sbuf, dbuf, sem):
            pltpu.make_async_copy(src_hbm.at[pl.ds(b, CHUNK)], sbuf, sem).wait()
            pltpu.make_async_copy(dst_hbm.at[pl.ds(b, CHUNK)], dbuf, sem).wait()

        load_idx(base, sidx0, didx0, is0)
        wait_idx(base, sidx0, didx0, is0)
        pltpu.async_copy(g_hbm.at[sidx0], rows0, gs0)
        load_idx(base + CHUNK, sidx1, didx1, is1)

        @pl.when(s < ZT)
        def _():
            pltpu.sync_copy(z_hbm.at[pl.ds(c * N + s * ZR, ZR)],
                            acc.at[pl.ds(s * ZR, ZR)])

        plsc.subcore_barrier()

        @pl.loop(0, nchunk // 2)
        def _(j):
            b0 = base + (2 * j) * CHUNK
            b1 = b0 + CHUNK
            wait_idx(b1, sidx1, didx1, is1)
            pltpu.async_copy(g_hbm.at[sidx1], rows1, gs1)
            pltpu.make_async_copy(g_hbm.at[sidx0], rows0, gs0).wait()
            pltpu.sync_copy(rows0, acc.at[didx0], add=True)

            @pl.when(2 * j + 2 < nchunk)
            def _():
                load_idx(b0 + 2 * CHUNK, sidx0, didx0, is0)

            pltpu.make_async_copy(g_hbm.at[sidx1], rows1, gs1).wait()
            pltpu.sync_copy(rows1, acc.at[didx1], add=True)

            @pl.when(2 * j + 3 < nchunk)
            def _():
                load_idx(b1 + 2 * CHUNK, sidx1, didx1, is1)

            @pl.when(2 * j + 2 < nchunk)
            def _():
                wait_idx(b0 + 2 * CHUNK, sidx0, didx0, is0)
                pltpu.async_copy(g_hbm.at[sidx0], rows0, gs0)

        plsc.subcore_barrier()

        @pl.when(s < ZT)
        def _():
            pltpu.sync_copy(acc.at[pl.ds(s * ZR, ZR)],
                            out_hbm.at[pl.ds(c * N + s * ZR, ZR)])

    return run(g, src1d, dst1d, zrows)


R = 1000
G = N // R


def _row_spec():
    return pl.BlockSpec((R, D), lambda i: (i, 0))


def _deg_spec():
    return pl.BlockSpec((R, 2), lambda i: (i, 0))


def _w_spec():
    return pl.BlockSpec((D, D), lambda i: (0, 0))


def _b_spec():
    return pl.BlockSpec((1, D), lambda i: (0, 0))


def _dinv_of(deg_blk):
    return lax.rsqrt(deg_blk[:, 0] + deg_blk[:, 1] + 1.0)


def _tc_pre(deg2, x, w):
    def body(deg_ref, x_ref, w_ref, g_ref):
        dinv = _dinv_of(deg_ref[...])
        h = jnp.dot(x_ref[...], w_ref[...], preferred_element_type=jnp.float32)
        g_ref[...] = h * dinv[:, None]

    return pl.pallas_call(
        body,
        grid=(G,),
        in_specs=[_deg_spec(), _row_spec(), _w_spec()],
        out_specs=_row_spec(),
        out_shape=jax.ShapeDtypeStruct((N, D), jnp.float32),
    )(deg2, x, w)


def _tc_mid(deg2, parts, gprev, b, w):
    def body(deg_ref, p0_ref, p1_ref, gp_ref, b_ref, w_ref, g_ref):
        dinv = _dinv_of(deg_ref[...])
        t = (p0_ref[...] + p1_ref[...] + gp_ref[...]) * dinv[:, None] + b_ref[...]
        z = jnp.maximum(t, 0.0)
        h = jnp.dot(z, w_ref[...], preferred_element_type=jnp.float32)
        g_ref[...] = h * dinv[:, None]

    p1_spec = pl.BlockSpec((R, D), lambda i: (G + i, 0))
    return pl.pallas_call(
        body,
        grid=(G,),
        in_specs=[_deg_spec(), _row_spec(), p1_spec, _row_spec(), _b_spec(), _w_spec()],
        out_specs=_row_spec(),
        out_shape=jax.ShapeDtypeStruct((N, D), jnp.float32),
    )(deg2, parts, parts, gprev, b, w)


def _tc_post(deg2, parts, g3, b):
    def body(deg_ref, p0_ref, p1_ref, g_ref, b_ref, o_ref):
        dinv = _dinv_of(deg_ref[...])
        o_ref[...] = (p0_ref[...] + p1_ref[...] + g_ref[...]) * dinv[:, None] + b_ref[...]

    p1_spec = pl.BlockSpec((R, D), lambda i: (G + i, 0))
    return pl.pallas_call(
        body,
        grid=(G,),
        in_specs=[_deg_spec(), _row_spec(), p1_spec, _row_spec(), _b_spec()],
        out_specs=_row_spec(),
        out_shape=jax.ShapeDtypeStruct((N, D), jnp.float32),
    )(deg2, parts, parts, g3, b)


def kernel(x, edge_index, W1, b1, W2, b2, W3, b3):
    ei = edge_index.astype(jnp.int32)
    npad = E_PAD - E
    src1d = jnp.concatenate([ei[0], jnp.zeros((npad,), jnp.int32)])
    dst1d = jnp.concatenate([ei[1], jnp.full((npad,), N, jnp.int32)])
    zdeg = jnp.zeros((NC * ND_PAD,), jnp.float32)
    zrows = jnp.zeros((NC * N, D), jnp.float32)

    degp = _sc_degree(dst1d, zdeg)
    deg2 = degp.reshape(NC, ND_PAD)[:, :N].T

    b1r = b1.reshape(1, D)
    b2r = b2.reshape(1, D)
    b3r = b3.reshape(1, D)

    g1 = _tc_pre(deg2, x, W1)
    p = _sc_aggregate(g1, src1d, dst1d, zrows)
    g2 = _tc_mid(deg2, p, g1, b1r, W2)
    q = _sc_aggregate(g2, src1d, dst1d, zrows)
    g3 = _tc_mid(deg2, q, g2, b2r, W3)
    r = _sc_aggregate(g3, src1d, dst1d, zrows)
    return _tc_post(deg2, r, g3, b3r)


# speedup vs baseline: 1.7673x; 1.4138x over previous
import functools

import jax
import jax.numpy as jnp
from jax import lax
from jax.experimental import pallas as pl
from jax.experimental.pallas import tpu as pltpu
from jax.experimental.pallas import tpu_sc as plsc

N = 10000
D = 128
E = 320000
NC = 2
NS = 16
NW = NC * NS
CHUNK = 88
NCHUNK0 = 148
NCHUNK1 = 80
EPT0 = CHUNK * NCHUNK0
EPT1 = CHUNK * NCHUNK1
E_PAD = NS * (EPT0 + EPT1)
N_ACC = N + 8
ZT = 10
ZR = N // ZT
ND_PAD = 10240
DPT = ND_PAD // NS


def _sc_mesh():
    return plsc.VectorSubcoreMesh(core_axis_name="c", subcore_axis_name="s")


def _sc_degree(dst1d, zdeg):
    @functools.partial(
        pl.kernel,
        out_type=jax.ShapeDtypeStruct((NC * ND_PAD,), jnp.float32),
        mesh=_sc_mesh(),
        scratch_types=[
            [pltpu.VMEM((CHUNK,), jnp.int32) for _ in range(4)],
            pltpu.VMEM((CHUNK,), jnp.float32),
            pltpu.VMEM_SHARED((ND_PAD,), jnp.float32),
            pltpu.SemaphoreType.DMA,
            pltpu.SemaphoreType.DMA,
        ],
    )
    def run(dst_hbm, z_hbm, out_hbm, dbufs, ones, acc, isem, ssem):
        c = lax.axis_index("c")
        s = lax.axis_index("s")
        base = jnp.where(c == 0, s * EPT0, NS * EPT0 + s * EPT1)
        ngrp = jnp.where(c == 0, NCHUNK0 // 4, NCHUNK1 // 4)
        for k in range(4):
            pltpu.async_copy(dst_hbm.at[pl.ds(base + k * CHUNK, CHUNK)],
                             dbufs[k], isem)
        pltpu.sync_copy(z_hbm.at[pl.ds(c * ND_PAD + s * DPT, DPT)],
                        acc.at[pl.ds(s * DPT, DPT)])
        for j in range(CHUNK // 16):
            ones[pl.ds(j * 16, 16)] = jnp.full((16,), 1.0, jnp.float32)
        plsc.subcore_barrier()

        @pl.loop(0, ngrp)
        def _(j):
            b = base + (4 * j) * CHUNK
            for k in range(4):
                pltpu.make_async_copy(dst_hbm.at[pl.ds(b + k * CHUNK, CHUNK)],
                                      dbufs[k], isem).wait()
                pltpu.async_copy(ones, acc.at[dbufs[k]], ssem, add=True)
            for k in range(4):
                pltpu.make_async_copy(ones, acc.at[dbufs[k]], ssem).wait()

            @pl.when(j + 1 < ngrp)
            def _():
                for k in range(4):
                    pltpu.async_copy(
                        dst_hbm.at[pl.ds(b + (4 + k) * CHUNK, CHUNK)],
                        dbufs[k], isem)

        plsc.subcore_barrier()
        pltpu.sync_copy(acc.at[pl.ds(s * DPT, DPT)],
                        out_hbm.at[pl.ds(c * ND_PAD + s * DPT, DPT)])

    return run(dst1d, zdeg)


def _sc_aggregate(g, src1d, dst1d, zrows):
    @functools.partial(
        pl.kernel,
        out_type=jax.ShapeDtypeStruct((NC * N, D), jnp.float32),
        mesh=_sc_mesh(),
        scratch_types=[
            pltpu.VMEM((CHUNK,), jnp.int32),
            pltpu.VMEM((CHUNK,), jnp.int32),
            pltpu.VMEM((CHUNK,), jnp.int32),
            pltpu.VMEM((CHUNK,), jnp.int32),
            pltpu.VMEM((CHUNK, D), jnp.float32),
            pltpu.VMEM((CHUNK, D), jnp.float32),
            pltpu.VMEM_SHARED((N_ACC, D), jnp.float32),
            pltpu.SemaphoreType.DMA,
            pltpu.SemaphoreType.DMA,
            pltpu.SemaphoreType.DMA,
            pltpu.SemaphoreType.DMA,
        ],
    )
    def run(g_hbm, src_hbm, dst_hbm, z_hbm, out_hbm,
            sidx0, didx0, sidx1, didx1, rows0, rows1, acc, is0, is1, gs0, gs1):
        c = lax.axis_index("c")
        s = lax.axis_index("s")
        base = jnp.where(c == 0, s * EPT0, NS * EPT0 + s * EPT1)
        nchunk = jnp.where(c == 0, NCHUNK0, NCHUNK1)

        def load_idx(b, sbuf, dbuf, sem):
            pltpu.async_copy(src_hbm.at[pl.ds(b, CHUNK)], sbuf, sem)
            pltpu.async_copy(dst_hbm.at[pl.ds(b, CHUNK)], dbuf, sem)

        def wait_idx(b, sbuf, dbuf, sem):
            pltpu.make_async_copy(src_hbm.at[pl.ds(b, CHUNK)], sbuf, sem).wait()
            pltpu.make_async_copy(dst_hbm.at[pl.ds(b, CHUNK)], dbuf, sem).wait()

        load_idx(base, sidx0, didx0, is0)
        wait_idx(base, sidx0, didx0, is0)
        pltpu.async_copy(g_hbm.at[sidx0], rows0, gs0)
        load_idx(base + CHUNK, sidx1, didx1, is1)

        @pl.when(s < ZT)
        def _():
            pltpu.sync_copy(z_hbm.at[pl.ds(c * N + s * ZR, ZR)],
                            acc.at[pl.ds(s * ZR, ZR)])

        plsc.subcore_barrier()

        @pl.loop(0, nchunk // 2)
        def _(j):
            b0 = base + (2 * j) * CHUNK
            b1 = b0 + CHUNK
            wait_idx(b1, sidx1, didx1, is1)
            pltpu.async_copy(g_hbm.at[sidx1], rows1, gs1)
            pltpu.make_async_copy(g_hbm.at[sidx0], rows0, gs0).wait()
            pltpu.sync_copy(rows0, acc.at[didx0], add=True)

            @pl.when(2 * j + 2 < nchunk)
            def _():
                load_idx(b0 + 2 * CHUNK, sidx0, didx0, is0)

            pltpu.make_async_copy(g_hbm.at[sidx1], rows1, gs1).wait()
            pltpu.sync_copy(rows1, acc.at[didx1], add=True)

            @pl.when(2 * j + 3 < nchunk)
            def _():
                load_idx(b1 + 2 * CHUNK, sidx1, didx1, is1)

            @pl.when(2 * j + 2 < nchunk)
            def _():
                wait_idx(b0 + 2 * CHUNK, sidx0, didx0, is0)
                pltpu.async_copy(g_hbm.at[sidx0], rows0, gs0)

        plsc.subcore_barrier()

        @pl.when(s < ZT)
        def _():
            pltpu.sync_copy(acc.at[pl.ds(s * ZR, ZR)],
                            out_hbm.at[pl.ds(c * N + s * ZR, ZR)])

    return run(g, src1d, dst1d, zrows)


R = 1000
G = N // R


def _row_spec():
    return pl.BlockSpec((R, D), lambda i: (i, 0))


def _deg_spec():
    return pl.BlockSpec((R, 2), lambda i: (i, 0))


def _w_spec():
    return pl.BlockSpec((D, D), lambda i: (0, 0))


def _b_spec():
    return pl.BlockSpec((1, D), lambda i: (0, 0))


def _dinv_of(deg_blk):
    return lax.rsqrt(deg_blk[:, 0] + deg_blk[:, 1] + 1.0)


def _tc_pre(deg2, x, w):
    def body(deg_ref, x_ref, w_ref, g_ref):
        dinv = _dinv_of(deg_ref[...])
        h = jnp.dot(x_ref[...], w_ref[...], preferred_element_type=jnp.float32)
        g_ref[...] = h * dinv[:, None]

    return pl.pallas_call(
        body,
        grid=(G,),
        in_specs=[_deg_spec(), _row_spec(), _w_spec()],
        out_specs=_row_spec(),
        out_shape=jax.ShapeDtypeStruct((N, D), jnp.float32),
    )(deg2, x, w)


def _tc_mid(deg2, parts, gprev, b, w):
    def body(deg_ref, p0_ref, p1_ref, gp_ref, b_ref, w_ref, g_ref):
        dinv = _dinv_of(deg_ref[...])
        t = (p0_ref[...] + p1_ref[...] + gp_ref[...]) * dinv[:, None] + b_ref[...]
        z = jnp.maximum(t, 0.0)
        h = jnp.dot(z, w_ref[...], preferred_element_type=jnp.float32)
        g_ref[...] = h * dinv[:, None]

    p1_spec = pl.BlockSpec((R, D), lambda i: (G + i, 0))
    return pl.pallas_call(
        body,
        grid=(G,),
        in_specs=[_deg_spec(), _row_spec(), p1_spec, _row_spec(), _b_spec(), _w_spec()],
        out_specs=_row_spec(),
        out_shape=jax.ShapeDtypeStruct((N, D), jnp.float32),
    )(deg2, parts, parts, gprev, b, w)


def _tc_post(deg2, parts, g3, b):
    def body(deg_ref, p0_ref, p1_ref, g_ref, b_ref, o_ref):
        dinv = _dinv_of(deg_ref[...])
        o_ref[...] = (p0_ref[...] + p1_ref[...] + g_ref[...]) * dinv[:, None] + b_ref[...]

    p1_spec = pl.BlockSpec((R, D), lambda i: (G + i, 0))
    return pl.pallas_call(
        body,
        grid=(G,),
        in_specs=[_deg_spec(), _row_spec(), p1_spec, _row_spec(), _b_spec()],
        out_specs=_row_spec(),
        out_shape=jax.ShapeDtypeStruct((N, D), jnp.float32),
    )(deg2, parts, parts, g3, b)


def kernel(x, edge_index, W1, b1, W2, b2, W3, b3):
    ei = edge_index.astype(jnp.int32)
    npad = E_PAD - E
    src1d = jnp.concatenate([ei[0], jnp.zeros((npad,), jnp.int32)])
    dst1d = jnp.concatenate([ei[1], jnp.full((npad,), N, jnp.int32)])
    zdeg = jnp.zeros((NC * ND_PAD,), jnp.float32)
    zrows = jnp.zeros((NC * N, D), jnp.float32)

    degp = _sc_degree(dst1d, zdeg)
    deg2 = degp.reshape(NC, ND_PAD)[:, :N].T

    b1r = b1.reshape(1, D)
    b2r = b2.reshape(1, D)
    b3r = b3.reshape(1, D)

    g1 = _tc_pre(deg2, x, W1)
    p = _sc_aggregate(g1, src1d, dst1d, zrows)
    g2 = _tc_mid(deg2, p, g1, b1r, W2)
    q = _sc_aggregate(g2, src1d, dst1d, zrows)
    g3 = _tc_mid(deg2, q, g2, b2r, W3)
    r = _sc_aggregate(g3, src1d, dst1d, zrows)
    return _tc_post(deg2, r, g3, b3r)
